# Initial kernel scaffold; baseline (speedup 1.0000x reference)
import jax
import jax.numpy as jnp
from jax.experimental import pallas as pl


def kernel(tokens, table):
    raise NotImplementedError("write your pallas kernel here")


# speedup vs baseline: 1.2925x; 1.2925x over previous
import math

import jax
import jax.numpy as jnp
from jax import lax
from jax.experimental import pallas as pl
from jax.experimental.pallas import tpu as pltpu
from jax.experimental.pallas import tpu_sc as plsc

_EMB = 32
_SCALE = math.sqrt(_EMB)
_NC, _NS = 2, 16
_NW = _NC * _NS
_IDX_MINOR = 128
_CHUNK = 1024
_GPC = _CHUNK // _IDX_MINOR


def _body(idx_hbm, table_hbm, out_hbm, idx_v, rows_v, gsem):
    wid = lax.axis_index("s") * _NC + lax.axis_index("c")
    nrows = idx_hbm.shape[0] * _IDX_MINOR
    rpw = nrows // _NW
    steps = rpw // _CHUNK
    irow0 = wid * (rpw // _IDX_MINOR)

    def step(g, carry):
        base = wid * rpw + g * _CHUNK
        pltpu.sync_copy(idx_hbm.at[pl.ds(irow0 + g * _GPC, _GPC)], idx_v)
        copies = []
        for j in range(_GPC):
            copies.append(pltpu.async_copy(
                table_hbm.at[idx_v.at[j]],
                rows_v.at[pl.ds(j * _IDX_MINOR, _IDX_MINOR)],
                gsem))
        for cp in copies:
            cp.wait()

        def scale_row(r, c2):
            rows_v[r, pl.ds(0, 16)] = rows_v[r, pl.ds(0, 16)] * _SCALE
            rows_v[r, pl.ds(16, 16)] = rows_v[r, pl.ds(16, 16)] * _SCALE
            return c2

        lax.fori_loop(0, _CHUNK, scale_row, 0)
        pltpu.sync_copy(rows_v, out_hbm.at[pl.ds(base, _CHUNK)])
        return carry

    lax.fori_loop(0, steps, step, 0)


@jax.jit
def _gather(idx2d, table):
    n = idx2d.shape[0] * idx2d.shape[1]
    mesh = plsc.VectorSubcoreMesh(core_axis_name="c", subcore_axis_name="s",
                                  num_cores=_NC, num_subcores=_NS)
    f = pl.kernel(
        _body,
        out_type=jax.ShapeDtypeStruct((n, _EMB), jnp.float32),
        mesh=mesh,
        compiler_params=pltpu.CompilerParams(use_tc_tiling_on_sc=False),
        scratch_types=[
            pltpu.VMEM((_GPC, _IDX_MINOR), jnp.int32),
            pltpu.VMEM((_CHUNK, _EMB), jnp.float32),
            pltpu.SemaphoreType.DMA,
        ],
    )
    return f(idx2d, table)


def kernel(tokens, table):
    b, s = tokens.shape
    idx2d = tokens.astype(jnp.int32).reshape(-1, _IDX_MINOR)
    out = _gather(idx2d, table)
    return out.reshape(b, s, _EMB)


# speedup vs baseline: 1.4376x; 1.1122x over previous
import math

import jax
import jax.numpy as jnp
from jax import lax
from jax.experimental import pallas as pl
from jax.experimental.pallas import tpu as pltpu
from jax.experimental.pallas import tpu_sc as plsc

_EMB = 32
_SCALE = math.sqrt(_EMB)
_NC, _NS = 2, 16
_NW = _NC * _NS
_IDX_MINOR = 128
_CHUNK = 512
_GPC = _CHUNK // _IDX_MINOR
_ROWS_PER_IT = 4


def _body(idx_hbm, table_hbm, out_hbm, idx_v, rows_v, gsem):
    wid = lax.axis_index("s") * _NC + lax.axis_index("c")
    nrows = idx_hbm.shape[0] * _IDX_MINOR
    rpw = nrows // _NW
    steps = rpw // _CHUNK
    irow0 = wid * (rpw // _IDX_MINOR)

    def load_idx(g, b):
        pltpu.sync_copy(idx_hbm.at[pl.ds(irow0 + g * _GPC, _GPC)], idx_v.at[b])

    def fire_gather(b):
        for j in range(_GPC):
            pltpu.async_copy(table_hbm.at[idx_v.at[b, j]],
                             rows_v.at[b, pl.ds(j * _IDX_MINOR, _IDX_MINOR)],
                             gsem)

    def wait_gather(b):
        for j in range(_GPC):
            pltpu.make_async_copy(
                table_hbm.at[idx_v.at[b, j]],
                rows_v.at[b, pl.ds(j * _IDX_MINOR, _IDX_MINOR)],
                gsem).wait()

    def scale(b):
        @pl.loop(0, _CHUNK // _ROWS_PER_IT, unroll=2)
        def _(r):
            for u in range(_ROWS_PER_IT):
                row = r * _ROWS_PER_IT + u
                rows_v[b, row, pl.ds(0, 16)] = rows_v[b, row, pl.ds(0, 16)] * _SCALE
                rows_v[b, row, pl.ds(16, 16)] = rows_v[b, row, pl.ds(16, 16)] * _SCALE

    def out_copy(g, b):
        pltpu.sync_copy(rows_v.at[b], out_hbm.at[pl.ds(wid * rpw + g * _CHUNK, _CHUNK)])

    load_idx(0, 0)
    fire_gather(0)

    @pl.loop(0, steps // 2)
    def _(t):
        g0 = 2 * t
        load_idx(g0 + 1, 1)
        fire_gather(1)
        wait_gather(0)
        scale(0)
        out_copy(g0, 0)

        @pl.when(g0 + 2 < steps)
        def _():
            load_idx(g0 + 2, 0)
            fire_gather(0)
        wait_gather(1)
        scale(1)
        out_copy(g0 + 1, 1)


@jax.jit
def _gather(idx2d, table):
    n = idx2d.shape[0] * idx2d.shape[1]
    mesh = plsc.VectorSubcoreMesh(core_axis_name="c", subcore_axis_name="s",
                                  num_cores=_NC, num_subcores=_NS)
    f = pl.kernel(
        _body,
        out_type=jax.ShapeDtypeStruct((n, _EMB), jnp.float32),
        mesh=mesh,
        compiler_params=pltpu.CompilerParams(use_tc_tiling_on_sc=False),
        scratch_types=[
            pltpu.VMEM((2, _GPC, _IDX_MINOR), jnp.int32),
            pltpu.VMEM((2, _CHUNK, _EMB), jnp.float32),
            pltpu.SemaphoreType.DMA,
        ],
    )
    return f(idx2d, table)


def kernel(tokens, table):
    b, s = tokens.shape
    idx2d = tokens.astype(jnp.int32).reshape(-1, _IDX_MINOR)
    out = _gather(idx2d, table)
    return out.reshape(b, s, _EMB)


# speedup vs baseline: 1.4556x; 1.0126x over previous
import math

import jax
import jax.numpy as jnp
from jax import lax
from jax.experimental import pallas as pl
from jax.experimental.pallas import tpu as pltpu
from jax.experimental.pallas import tpu_sc as plsc

_EMB = 32
_SCALE = math.sqrt(_EMB)
_NC, _NS = 2, 16
_NW = _NC * _NS
_SEQ = 200
_ROWS_PER_CHUNK = 4
_SPLITS = ((0, 128), (128, 72))


def _body(idx_hbm, table_hbm, out_hbm, idx_v, rows_v, gsem):
    wid = lax.axis_index("s") * _NC + lax.axis_index("c")
    rows_total = idx_hbm.shape[0]
    rpw = rows_total // _NW
    steps = rpw // _ROWS_PER_CHUNK
    row0 = wid * rpw

    def load_idx(g, b):
        pltpu.sync_copy(idx_hbm.at[pl.ds(row0 + g * _ROWS_PER_CHUNK,
                                         _ROWS_PER_CHUNK)], idx_v.at[b])

    def gather_descs(b):
        for r in range(_ROWS_PER_CHUNK):
            for off, width in _SPLITS:
                yield (table_hbm.at[idx_v.at[b, r, pl.ds(off, width)]],
                       rows_v.at[b, r, pl.ds(off, width)])

    def fire_gather(b):
        for src, dst in gather_descs(b):
            pltpu.async_copy(src, dst, gsem)

    def wait_gather(b):
        for src, dst in gather_descs(b):
            pltpu.make_async_copy(src, dst, gsem).wait()

    def scale(b):
        for r in range(_ROWS_PER_CHUNK):
            @pl.loop(0, _SEQ // 2, unroll=2)
            def _(c2):
                for u in range(2):
                    c = c2 * 2 + u
                    rows_v[b, r, c, pl.ds(0, 16)] = (
                        rows_v[b, r, c, pl.ds(0, 16)] * _SCALE)
                    rows_v[b, r, c, pl.ds(16, 16)] = (
                        rows_v[b, r, c, pl.ds(16, 16)] * _SCALE)

    def out_copy(g, b):
        pltpu.sync_copy(rows_v.at[b],
                        out_hbm.at[pl.ds(row0 + g * _ROWS_PER_CHUNK,
                                         _ROWS_PER_CHUNK)])

    load_idx(0, 0)
    fire_gather(0)

    @pl.loop(0, steps // 2)
    def _(t):
        g0 = 2 * t
        load_idx(g0 + 1, 1)
        fire_gather(1)
        wait_gather(0)
        scale(0)
        out_copy(g0, 0)

        @pl.when(g0 + 2 < steps)
        def _():
            load_idx(g0 + 2, 0)
            fire_gather(0)
        wait_gather(1)
        scale(1)
        out_copy(g0 + 1, 1)


@jax.jit
def _gather(tokens, table):
    rows, seq = tokens.shape
    mesh = plsc.VectorSubcoreMesh(core_axis_name="c", subcore_axis_name="s",
                                  num_cores=_NC, num_subcores=_NS)
    f = pl.kernel(
        _body,
        out_type=jax.ShapeDtypeStruct((rows, seq, _EMB), jnp.float32),
        mesh=mesh,
        compiler_params=pltpu.CompilerParams(use_tc_tiling_on_sc=False),
        scratch_types=[
            pltpu.VMEM((2, _ROWS_PER_CHUNK, _SEQ), jnp.int32),
            pltpu.VMEM((2, _ROWS_PER_CHUNK, _SEQ, _EMB), jnp.float32),
            pltpu.SemaphoreType.DMA,
        ],
    )
    return f(tokens, table)


def kernel(tokens, table):
    return _gather(tokens.astype(jnp.int32), table)
